# baseline (speedup 1.0000x reference)
import functools

import jax
import jax.numpy as jnp
from jax import lax
from jax.experimental import pallas as pl
from jax.experimental.pallas import tpu as pltpu
from jax.experimental.pallas import tpu_sc as plsc

N_NODES = 10000
N_PAD = 10240
NC, NS = 2, 16
NW = NC * NS
E_EDGES = 320000
EDGES_PER_TILE = E_EDGES // NW
ROWS_PER_TILE = N_PAD // NS
LAST_ROWS = N_NODES - (NS - 1) * ROWS_PER_TILE


def _lrelu(v):
    return jnp.where(v >= 0, v, v * 0.01)


@functools.lru_cache(maxsize=None)
def _make_sc_scatter(d, chunk, ring, depth):
    n_chunks = EDGES_PER_TILE // chunk
    assert n_chunks % ring == 0 and depth < ring
    mesh = plsc.VectorSubcoreMesh(
        core_axis_name="c", subcore_axis_name="s", num_cores=NC, num_subcores=NS
    )

    @functools.partial(
        pl.kernel,
        out_type=jax.ShapeDtypeStruct((NC * N_NODES, d), jnp.float32),
        mesh=mesh,
        scratch_types=(
            [pltpu.VMEM_SHARED((N_PAD, d), jnp.float32)]
            + [pltpu.VMEM((n_chunks, chunk), jnp.int32)] * 2
            + [pltpu.VMEM((chunk, d), jnp.float32)] * ring
            + [pltpu.SemaphoreType.DMA] * (2 * ring)
        ),
        compiler_params=pltpu.CompilerParams(use_tc_tiling_on_sc=False),
    )
    def sc_scatter(x_hbm, src_hbm, dst_hbm, zeros_hbm, out_hbm,
                   acc, src_v, dst_v, *bufs):
        rows = bufs[:ring]
        gsem = bufs[ring:2 * ring]
        ssem = bufs[2 * ring:]
        c = lax.axis_index("c")
        s = lax.axis_index("s")
        wid = s * NC + c
        row0 = s * ROWS_PER_TILE
        pltpu.sync_copy(zeros_hbm, rows[0])
        for g in range(ROWS_PER_TILE // chunk):
            pltpu.sync_copy(rows[0], acc.at[pl.ds(row0 + g * chunk, chunk)])
        pltpu.sync_copy(src_hbm.at[wid], src_v)
        pltpu.sync_copy(dst_hbm.at[wid], dst_v)
        plsc.subcore_barrier()

        def gather(j, b):
            pltpu.async_copy(x_hbm.at[src_v.at[j]], rows[b], gsem[b])

        def scatter(j, b):
            pltpu.async_copy(rows[b], acc.at[dst_v.at[j]], ssem[b],
                             add=True)

        def gwait(b):
            pltpu.make_async_copy(x_hbm.at[src_v.at[0]], rows[b],
                                  gsem[b]).wait()

        def swait(b):
            pltpu.make_async_copy(rows[b], acc.at[dst_v.at[0]],
                                  ssem[b]).wait()

        def round_body(i, carry):
            for k in range(ring):
                j = i * ring + k
                b2 = (k - depth) % ring
                @pl.when(i > 0)
                def _():
                    swait(k)
                gather(j, k)
                if k >= depth:
                    gwait(b2)
                    scatter(j - depth, b2)
                else:
                    @pl.when(i > 0)
                    def _():
                        gwait(b2)
                        scatter(j - depth, b2)
            return carry

        lax.fori_loop(0, n_chunks // ring, round_body, 0)
        for k in range(depth):
            j = n_chunks - depth + k
            b = j % ring
            gwait(b)
            scatter(j, b)
        for k in range(ring):
            swait(k)
        plsc.subcore_barrier()
        @pl.when(s < NS - 1)
        def _():
            pltpu.sync_copy(acc.at[pl.ds(row0, ROWS_PER_TILE)],
                            out_hbm.at[pl.ds(c * N_NODES + row0,
                                             ROWS_PER_TILE)])

        @pl.when(s == NS - 1)
        def _():
            pltpu.sync_copy(acc.at[pl.ds(row0, LAST_ROWS)],
                            out_hbm.at[pl.ds(c * N_NODES + row0,
                                             LAST_ROWS)])

    return sc_scatter


_RB = 1000
_GRID = (N_NODES // _RB,)


def _dotT(a, w):
    return lax.dot_general(a, w, (((1,), (1,)), ((), ())),
                           preferred_element_type=jnp.float32)


def _tc_pre_body(x_ref, lin1_w_ref, lin1_b_ref, id_ref, xn_ref, xh_ref):
    xb = x_ref[...]
    nrm = jnp.sqrt(jnp.sum(xb * xb, axis=1, keepdims=True))
    xn = xb / jnp.maximum(nrm, 1e-12)
    xn_ref[...] = xn
    y = _dotT(xn, lin1_w_ref[...]) + lin1_b_ref[...]
    xh_ref[...] = _lrelu(y) + id_ref[...]


def _tc_mid_body(s0_ref, s1_ref, conv_w1_ref, g1_w_ref, g1_b_ref, xh1_ref,
                 lin2_w_ref, lin2_b_ref, id_ref, x2_ref, xh2_ref):
    sacc = s0_ref[...] + s1_ref[...]
    h = _lrelu(jnp.dot(sacc, conv_w1_ref[...],
                       preferred_element_type=jnp.float32))
    x2 = _lrelu(_dotT(h, g1_w_ref[...]) + g1_b_ref[...] + xh1_ref[...])
    x2_ref[...] = x2
    y = _dotT(x2, lin2_w_ref[...]) + lin2_b_ref[...]
    xh2_ref[...] = _lrelu(y) + id_ref[...]


def _tc_post_body(t0_ref, t1_ref, conv_w2_ref, g2_w_ref, g2_b_ref, xh2_ref,
                  out_ref):
    tacc = t0_ref[...] + t1_ref[...]
    h = _lrelu(jnp.dot(tacc, conv_w2_ref[...],
                       preferred_element_type=jnp.float32))
    out_ref[...] = _lrelu(_dotT(h, g2_w_ref[...]) + g2_b_ref[...]
                          + xh2_ref[...])


def _row_spec(cols):
    return pl.BlockSpec((_RB, cols), lambda i: (i, 0))


def _part1_spec(cols):
    return pl.BlockSpec((_RB, cols), lambda i: (i + N_NODES // _RB, 0))


def _full_spec(r, cols):
    return pl.BlockSpec((r, cols), lambda i: (0, 0))


_tc_pre = pl.pallas_call(
    _tc_pre_body,
    grid=_GRID,
    in_specs=[_row_spec(128), _full_spec(64, 128), _full_spec(1, 64),
              _row_spec(64)],
    out_specs=[_row_spec(128), _row_spec(64)],
    out_shape=[jax.ShapeDtypeStruct((N_NODES, 128), jnp.float32),
               jax.ShapeDtypeStruct((N_NODES, 64), jnp.float32)],
)

_tc_mid = pl.pallas_call(
    _tc_mid_body,
    grid=_GRID,
    in_specs=[_row_spec(128), _part1_spec(128), _full_spec(128, 128),
              _full_spec(64, 128), _full_spec(1, 64), _row_spec(64),
              _full_spec(64, 64), _full_spec(1, 64), _row_spec(64)],
    out_specs=[_row_spec(64), _row_spec(64)],
    out_shape=[jax.ShapeDtypeStruct((N_NODES, 64), jnp.float32),
               jax.ShapeDtypeStruct((N_NODES, 64), jnp.float32)],
)

_tc_post = pl.pallas_call(
    _tc_post_body,
    grid=_GRID,
    in_specs=[_row_spec(64), _part1_spec(64), _full_spec(64, 64),
              _full_spec(64, 64), _full_spec(1, 64), _row_spec(64)],
    out_specs=_row_spec(64),
    out_shape=jax.ShapeDtypeStruct((N_NODES, 64), jnp.float32),
)


def kernel(features, id_embedding, preference, conv_w1, lin1_w, lin1_b,
           g1_w, g1_b, conv_w2, lin2_w, lin2_b, g2_w, g2_b, edge_index):
    src1 = edge_index[0].reshape(NW, EDGES_PER_TILE // 40, 40)
    dst1 = edge_index[1].reshape(NW, EDGES_PER_TILE // 40, 40)
    src2 = edge_index[0].reshape(NW, EDGES_PER_TILE // 80, 80)
    dst2 = edge_index[1].reshape(NW, EDGES_PER_TILE // 80, 80)

    zeros128 = jnp.zeros((40, 128), jnp.float32)
    zeros64 = jnp.zeros((80, 64), jnp.float32)

    xcat = jnp.concatenate([preference, features], axis=0)
    xn, xh1 = _tc_pre(xcat, lin1_w, lin1_b.reshape(1, 64), id_embedding)

    s_parts = _make_sc_scatter(128, 40, 5, 2)(xn, src1, dst1, zeros128)

    x2, xh2 = _tc_mid(s_parts, s_parts, conv_w1, g1_w, g1_b.reshape(1, 64),
                      xh1, lin2_w, lin2_b.reshape(1, 64), id_embedding)

    t_parts = _make_sc_scatter(64, 80, 5, 2)(x2, src2, dst2, zeros64)

    return _tc_post(t_parts, t_parts, conv_w2, g2_w, g2_b.reshape(1, 64),
                    xh2)


# speedup vs baseline: 1.0385x; 1.0385x over previous
import functools

import jax
import jax.numpy as jnp
from jax import lax
from jax.experimental import pallas as pl
from jax.experimental.pallas import tpu as pltpu
from jax.experimental.pallas import tpu_sc as plsc

N_NODES = 10000
N_PAD = 10016
NC, NS = 2, 16
NW = NC * NS
E_EDGES = 320000
EDGES_PER_TILE = E_EDGES // NW
ROWS_PER_TILE = N_PAD // NS
LAST_ROWS = N_NODES - (NS - 1) * ROWS_PER_TILE


def _lrelu(v):
    return jnp.where(v >= 0, v, v * 0.01)


@functools.lru_cache(maxsize=None)
def _make_sc_scatter(d, chunk, ring, depth):
    n_chunks = EDGES_PER_TILE // chunk
    assert n_chunks * chunk == EDGES_PER_TILE and depth < ring
    rounds = n_chunks // ring
    tail = n_chunks - rounds * ring
    assert rounds >= 2
    mesh = plsc.VectorSubcoreMesh(
        core_axis_name="c", subcore_axis_name="s", num_cores=NC, num_subcores=NS
    )

    @functools.partial(
        pl.kernel,
        out_type=jax.ShapeDtypeStruct((NC * N_NODES, d), jnp.float32),
        mesh=mesh,
        scratch_types=(
            [pltpu.VMEM_SHARED((N_PAD, d), jnp.float32)]
            + [pltpu.VMEM((n_chunks, chunk), jnp.int32)] * 2
            + [pltpu.VMEM((chunk, d), jnp.float32)] * ring
            + [pltpu.SemaphoreType.DMA] * (2 * ring)
        ),
        compiler_params=pltpu.CompilerParams(use_tc_tiling_on_sc=False),
    )
    def sc_scatter(x_hbm, src_hbm, dst_hbm, zeros_hbm, out_hbm,
                   acc, src_v, dst_v, *bufs):
        rows = bufs[:ring]
        gsem = bufs[ring:2 * ring]
        ssem = bufs[2 * ring:]
        c = lax.axis_index("c")
        s = lax.axis_index("s")
        wid = s * NC + c
        row0 = s * ROWS_PER_TILE
        pltpu.sync_copy(zeros_hbm, rows[0])
        for g in range(ROWS_PER_TILE // chunk):
            pltpu.sync_copy(rows[0], acc.at[pl.ds(row0 + g * chunk, chunk)])
        zrem = ROWS_PER_TILE % chunk
        if zrem:
            pltpu.sync_copy(
                rows[0].at[pl.ds(0, zrem)],
                acc.at[pl.ds(row0 + (ROWS_PER_TILE // chunk) * chunk, zrem)])
        pltpu.sync_copy(src_hbm.at[wid], src_v)
        pltpu.sync_copy(dst_hbm.at[wid], dst_v)
        plsc.subcore_barrier()

        def gather(j, b):
            pltpu.async_copy(x_hbm.at[src_v.at[j]], rows[b], gsem[b])

        def scatter(j, b):
            pltpu.async_copy(rows[b], acc.at[dst_v.at[j]], ssem[b],
                             add=True)

        def gwait(b):
            pltpu.make_async_copy(x_hbm.at[src_v.at[0]], rows[b],
                                  gsem[b]).wait()

        def swait(b):
            pltpu.make_async_copy(rows[b], acc.at[dst_v.at[0]],
                                  ssem[b]).wait()

        def round_body(i, carry):
            for k in range(ring):
                j = i * ring + k
                b2 = (k - depth) % ring
                @pl.when(i > 0)
                def _():
                    swait(k)
                gather(j, k)
                if k >= depth:
                    gwait(b2)
                    scatter(j - depth, b2)
                else:
                    @pl.when(i > 0)
                    def _():
                        gwait(b2)
                        scatter(j - depth, b2)
            return carry

        lax.fori_loop(0, rounds, round_body, 0)
        for t in range(tail):
            j = rounds * ring + t
            swait(t)
            gather(j, t)
            b2 = (t - depth) % ring
            gwait(b2)
            scatter(j - depth, b2)
        for k in range(depth):
            j = n_chunks - depth + k
            b = j % ring
            gwait(b)
            scatter(j, b)
        for k in range(ring):
            swait(k)
        plsc.subcore_barrier()
        @pl.when(s < NS - 1)
        def _():
            pltpu.sync_copy(acc.at[pl.ds(row0, ROWS_PER_TILE)],
                            out_hbm.at[pl.ds(c * N_NODES + row0,
                                             ROWS_PER_TILE)])

        @pl.when(s == NS - 1)
        def _():
            pltpu.sync_copy(acc.at[pl.ds(row0, LAST_ROWS)],
                            out_hbm.at[pl.ds(c * N_NODES + row0,
                                             LAST_ROWS)])

    return sc_scatter


_RB = 1000
_GRID = (N_NODES // _RB,)


def _dotT(a, w):
    return lax.dot_general(a, w, (((1,), (1,)), ((), ())),
                           preferred_element_type=jnp.float32)


def _tc_pre_body(x_ref, lin1_w_ref, lin1_b_ref, id_ref, xn_ref, xh_ref):
    xb = x_ref[...]
    nrm = jnp.sqrt(jnp.sum(xb * xb, axis=1, keepdims=True))
    xn = xb / jnp.maximum(nrm, 1e-12)
    xn_ref[...] = xn
    y = _dotT(xn, lin1_w_ref[...]) + lin1_b_ref[...]
    xh_ref[...] = _lrelu(y) + id_ref[...]


def _tc_mid_body(s0_ref, s1_ref, conv_w1_ref, g1_w_ref, g1_b_ref, xh1_ref,
                 lin2_w_ref, lin2_b_ref, id_ref, x2_ref, xh2_ref):
    sacc = s0_ref[...] + s1_ref[...]
    h = _lrelu(jnp.dot(sacc, conv_w1_ref[...],
                       preferred_element_type=jnp.float32))
    x2 = _lrelu(_dotT(h, g1_w_ref[...]) + g1_b_ref[...] + xh1_ref[...])
    x2_ref[...] = x2
    y = _dotT(x2, lin2_w_ref[...]) + lin2_b_ref[...]
    xh2_ref[...] = _lrelu(y) + id_ref[...]


def _tc_post_body(t0_ref, t1_ref, conv_w2_ref, g2_w_ref, g2_b_ref, xh2_ref,
                  out_ref):
    tacc = t0_ref[...] + t1_ref[...]
    h = _lrelu(jnp.dot(tacc, conv_w2_ref[...],
                       preferred_element_type=jnp.float32))
    out_ref[...] = _lrelu(_dotT(h, g2_w_ref[...]) + g2_b_ref[...]
                          + xh2_ref[...])


def _row_spec(cols):
    return pl.BlockSpec((_RB, cols), lambda i: (i, 0))


def _part1_spec(cols):
    return pl.BlockSpec((_RB, cols), lambda i: (i + N_NODES // _RB, 0))


def _full_spec(r, cols):
    return pl.BlockSpec((r, cols), lambda i: (0, 0))


_tc_pre = pl.pallas_call(
    _tc_pre_body,
    grid=_GRID,
    in_specs=[_row_spec(128), _full_spec(64, 128), _full_spec(1, 64),
              _row_spec(64)],
    out_specs=[_row_spec(128), _row_spec(64)],
    out_shape=[jax.ShapeDtypeStruct((N_NODES, 128), jnp.float32),
               jax.ShapeDtypeStruct((N_NODES, 64), jnp.float32)],
)

_tc_mid = pl.pallas_call(
    _tc_mid_body,
    grid=_GRID,
    in_specs=[_row_spec(128), _part1_spec(128), _full_spec(128, 128),
              _full_spec(64, 128), _full_spec(1, 64), _row_spec(64),
              _full_spec(64, 64), _full_spec(1, 64), _row_spec(64)],
    out_specs=[_row_spec(64), _row_spec(64)],
    out_shape=[jax.ShapeDtypeStruct((N_NODES, 64), jnp.float32),
               jax.ShapeDtypeStruct((N_NODES, 64), jnp.float32)],
)

_tc_post = pl.pallas_call(
    _tc_post_body,
    grid=_GRID,
    in_specs=[_row_spec(64), _part1_spec(64), _full_spec(64, 64),
              _full_spec(64, 64), _full_spec(1, 64), _row_spec(64)],
    out_specs=_row_spec(64),
    out_shape=jax.ShapeDtypeStruct((N_NODES, 64), jnp.float32),
)


def kernel(features, id_embedding, preference, conv_w1, lin1_w, lin1_b,
           g1_w, g1_b, conv_w2, lin2_w, lin2_b, g2_w, g2_b, edge_index):
    src = edge_index[0].reshape(NW, EDGES_PER_TILE // 80, 80)
    dst = edge_index[1].reshape(NW, EDGES_PER_TILE // 80, 80)

    zeros128 = jnp.zeros((80, 128), jnp.float32)
    zeros64 = jnp.zeros((80, 64), jnp.float32)

    xcat = jnp.concatenate([preference, features], axis=0)
    xn, xh1 = _tc_pre(xcat, lin1_w, lin1_b.reshape(1, 64), id_embedding)

    s_parts = _make_sc_scatter(128, 80, 3, 2)(xn, src, dst, zeros128)

    x2, xh2 = _tc_mid(s_parts, s_parts, conv_w1, g1_w, g1_b.reshape(1, 64),
                      xh1, lin2_w, lin2_b.reshape(1, 64), id_embedding)

    t_parts = _make_sc_scatter(64, 80, 5, 2)(x2, src, dst, zeros64)

    return _tc_post(t_parts, t_parts, conv_w2, g2_w, g2_b.reshape(1, 64),
                    xh2)
